# baseline (speedup 1.0000x reference)
import functools

import jax
import jax.numpy as jnp
from jax import lax
from jax.experimental import pallas as pl
from jax.experimental.pallas import tpu as pltpu
from jax.experimental.pallas import tpu_sc as plsc

_N_NODES = 10000
_D = 128
_N_REL = 16
_E = 320000

_L = 16
_NC = 2
_NS = 16
_NW = _NC * _NS
_EW = _E // _NW
_C = 40
_NCHUNK = _EW // _C
_DCH = _D // _L
_OPAD = _EW + _L


def _sc_body(node_hbm, idx_hbm, relw_hbm, out_hbm,
             node_sh,
             rtab_v, ibuf0, ibuf1,
             hrows0, trows0, hrows1, trows1,
             out_v,
             si0, si1, sh0, st0, sh1, st1):
    wid = lax.axis_index("s") * _NC + lax.axis_index("c")
    sid = lax.axis_index("s")
    lane = lax.iota(jnp.int32, _L)
    perms = [jnp.bitwise_xor(lane, jnp.int32(1 << b)) for b in range(4)]
    dcols = [lane + jnp.int32(dd * _L) for dd in range(_DCH)]
    gdn = lax.GatherDimensionNumbers(
        offset_dims=(), collapsed_slice_dims=(0,), start_index_map=(0,))

    def _permute(x, p):
        return lax.gather(x, p[:, None], gdn, (1,),
                          mode=lax.GatherScatterMode.PROMISE_IN_BOUNDS)

    ibufs = (ibuf0, ibuf1)
    isems = (si0, si1)
    rows = ((hrows0, trows0), (hrows1, trows1))
    gsems = ((sh0, st0), (sh1, st1))

    @pl.when(sid == 0)
    def _():
        pltpu.sync_copy(node_hbm, node_sh)

    pltpu.sync_copy(relw_hbm, rtab_v)
    plsc.subcore_barrier()

    def fire_idx(i, b):
        pltpu.async_copy(idx_hbm.at[wid, i], ibufs[b], isems[b])

    def wait_idx(i, b):
        pltpu.make_async_copy(idx_hbm.at[wid, i], ibufs[b], isems[b]).wait()

    def fire_g(b):
        hr, tr = rows[b]
        sh, st = gsems[b]
        ib = ibufs[b]
        pltpu.async_copy(node_sh.at[ib.at[0]], hr, sh)
        pltpu.async_copy(node_sh.at[ib.at[1]], tr, st)

    def wait_g(b):
        hr, tr = rows[b]
        sh, st = gsems[b]
        ib = ibufs[b]
        pltpu.make_async_copy(node_sh.at[ib.at[0]], hr, sh).wait()
        pltpu.make_async_copy(node_sh.at[ib.at[1]], tr, st).wait()

    def compute(i, b):
        hr, tr = rows[b]
        ib = ibufs[b]

        def edges(base_row, rvec, koff, nk, scores):
            for k in range(nk):
                row = base_row + k
                rel_e = rvec[k + koff]
                ps = [hr[row, pl.ds(dd * _L, _L)]
                      * rtab_v[rel_e, pl.ds(dd * _L, _L)]
                      * tr[row, pl.ds(dd * _L, _L)]
                      for dd in range(_DCH)]
                while len(ps) > 1:
                    ps = [ps[m] + ps[m + 1] for m in range(0, len(ps), 2)]
                acc = ps[0]
                for p in perms:
                    acc = acc + _permute(acc, p)
                scores = jnp.where(lane == k, acc, scores)
            return scores

        def group_body(g, _):
            rvec = ib[2, pl.ds(g * _L, _L)]
            scores = edges(g * _L, rvec, 0, _L, jnp.zeros((_L,), jnp.float32))
            out_v[pl.ds(i * _C + g * _L, _L)] = scores * (1.0 / _D)
            return 0

        lax.fori_loop(0, _C // _L, group_body, 0)
        tail_base = (_C // _L) * _L
        rvec_t = ib[2, pl.ds(_C - _L, _L)]
        scores = edges(tail_base, rvec_t, _L - (_C - tail_base), 8,
                       jnp.zeros((_L,), jnp.float32))
        out_v[pl.ds(i * _C + tail_base, _L)] = scores * (1.0 / _D)

    last = jnp.int32(_NCHUNK - 1)
    fire_idx(0, 0)
    fire_idx(1, 1)
    wait_idx(0, 0)
    fire_g(0)
    wait_idx(1, 1)
    fire_g(1)

    def pair_body(u, _):
        for b in range(2):
            i = u * 2 + b
            nxt = jnp.minimum(i + 2, last)
            wait_g(b)
            fire_idx(nxt, b)
            compute(i, b)
            wait_idx(nxt, b)
            fire_g(b)
        return 0

    lax.fori_loop(0, _NCHUNK // 2, pair_body, 0)
    wait_g(0)
    wait_g(1)
    pltpu.sync_copy(out_v, out_hbm.at[wid])


@jax.jit
def _sc_score(node_embeddings, idx_all, rel_weight):
    mesh = plsc.VectorSubcoreMesh(core_axis_name="c", subcore_axis_name="s")
    kfn = functools.partial(
        pl.kernel,
        mesh=mesh,
        out_type=jax.ShapeDtypeStruct((_NW, _OPAD), jnp.float32),
        scratch_types=[
            pltpu.VMEM_SHARED((_N_NODES, _D), jnp.float32),
            pltpu.VMEM((_N_REL, _D), jnp.float32),
            pltpu.VMEM((3, _C), jnp.int32),
            pltpu.VMEM((3, _C), jnp.int32),
            pltpu.VMEM((_C, _D), jnp.float32),
            pltpu.VMEM((_C, _D), jnp.float32),
            pltpu.VMEM((_C, _D), jnp.float32),
            pltpu.VMEM((_C, _D), jnp.float32),
            pltpu.VMEM((_OPAD,), jnp.float32),
            pltpu.SemaphoreType.DMA,
            pltpu.SemaphoreType.DMA,
            pltpu.SemaphoreType.DMA,
            pltpu.SemaphoreType.DMA,
            pltpu.SemaphoreType.DMA,
            pltpu.SemaphoreType.DMA,
        ],
    )(_sc_body)
    return kfn(node_embeddings, idx_all, rel_weight)


def kernel(node_embeddings, edge_index, relation_type, rel_weight):
    head = edge_index[0].reshape(_NW, _NCHUNK, _C)
    tail = edge_index[1].reshape(_NW, _NCHUNK, _C)
    rel_type = relation_type.astype(jnp.int32).reshape(_NW, _NCHUNK, _C)
    idx_all = jnp.stack([head, tail, rel_type], axis=2)
    out = _sc_score(node_embeddings, idx_all, rel_weight)
    return out[:, :_EW].reshape(_E)


# speedup vs baseline: 2.3121x; 2.3121x over previous
import functools

import jax
import jax.numpy as jnp
from jax import lax
from jax.experimental import pallas as pl
from jax.experimental.pallas import tpu as pltpu
from jax.experimental.pallas import tpu_sc as plsc

_N_NODES = 10000
_D = 128
_N_REL = 16
_E = 320000

_L = 16
_NC = 2
_NS = 16
_NW = _NC * _NS
_EW = _E // _NW
_C = 40
_NCHUNK = _EW // _C
_DCH = _D // _L
_OPAD = _EW + _L


def _sc_body(node_hbm, idx_hbm, relw_hbm, out_hbm,
             node_sh,
             rtab_v, relbuf_v, ibuf0, ibuf1,
             hrows0, trows0, hrows1, trows1,
             out_v,
             si0, si1, sh0, st0, sh1, st1):
    wid = lax.axis_index("s") * _NC + lax.axis_index("c")
    sid = lax.axis_index("s")
    lane = lax.iota(jnp.int32, _L)
    perms = [jnp.bitwise_xor(lane, jnp.int32(1 << b)) for b in range(4)]
    dcols = [lane + jnp.int32(dd * _L) for dd in range(_DCH)]
    gdn = lax.GatherDimensionNumbers(
        offset_dims=(), collapsed_slice_dims=(0,), start_index_map=(0,))

    def _permute(x, p):
        return lax.gather(x, p[:, None], gdn, (1,),
                          mode=lax.GatherScatterMode.PROMISE_IN_BOUNDS)

    ibufs = (ibuf0, ibuf1)
    isems = (si0, si1)
    rows = ((hrows0, trows0), (hrows1, trows1))
    gsems = ((sh0, st0), (sh1, st1))

    @pl.when(sid == 0)
    def _():
        pltpu.sync_copy(node_hbm, node_sh)

    pltpu.sync_copy(relw_hbm, rtab_v)
    plsc.subcore_barrier()

    def fire_idx(i, b):
        pltpu.async_copy(idx_hbm.at[wid, i], ibufs[b], isems[b])

    def wait_idx(i, b):
        pltpu.make_async_copy(idx_hbm.at[wid, i], ibufs[b], isems[b]).wait()

    def fire_g(b):
        hr, tr = rows[b]
        sh, st = gsems[b]
        ib = ibufs[b]
        pltpu.async_copy(node_sh.at[ib.at[0]], hr, sh)
        pltpu.async_copy(node_sh.at[ib.at[1]], tr, st)

    def wait_g(b):
        hr, tr = rows[b]
        sh, st = gsems[b]
        ib = ibufs[b]
        pltpu.make_async_copy(node_sh.at[ib.at[0]], hr, sh).wait()
        pltpu.make_async_copy(node_sh.at[ib.at[1]], tr, st).wait()

    def stash_rel(b):
        ib = ibufs[b]
        relbuf_v[pl.ds(0, _L)] = ib[2, pl.ds(0, _L)]
        relbuf_v[pl.ds(_L, _L)] = ib[2, pl.ds(_L, _L)]
        relbuf_v[pl.ds(_C - _L, _L)] = ib[2, pl.ds(_C - _L, _L)]

    def compute(i, b):
        hr, tr = rows[b]

        def quad(nk, base_row, q4, scores):
            rvq = relbuf_v[pl.ds(base_row + q4 * 4, _L)]
            for j in range(nk):
                k = q4 * 4 + j
                row = base_row + k
                rel_e = rvq[j]
                ps = [hr[row, pl.ds(dd * _L, _L)]
                      * rtab_v[rel_e, pl.ds(dd * _L, _L)]
                      * tr[row, pl.ds(dd * _L, _L)]
                      for dd in range(_DCH)]
                while len(ps) > 1:
                    ps = [ps[m] + ps[m + 1] for m in range(0, len(ps), 2)]
                acc = ps[0]
                for p in perms:
                    acc = acc + _permute(acc, p)
                scores = jnp.where(lane == k, acc, scores)
            return scores

        def group_body(g, _):
            scores = lax.fori_loop(
                0, 4, functools.partial(quad, 4, g * _L),
                jnp.zeros((_L,), jnp.float32))
            out_v[pl.ds(i * _C + g * _L, _L)] = scores * (1.0 / _D)
            return 0

        lax.fori_loop(0, _C // _L, group_body, 0)
        tail_base = (_C // _L) * _L
        scores = lax.fori_loop(
            0, 2, functools.partial(quad, 4, tail_base),
            jnp.zeros((_L,), jnp.float32))
        out_v[pl.ds(i * _C + tail_base, _L)] = scores * (1.0 / _D)

    last = jnp.int32(_NCHUNK - 1)
    fire_idx(0, 0)
    fire_idx(1, 1)
    wait_idx(0, 0)
    fire_g(0)
    wait_idx(1, 1)
    fire_g(1)

    def pair_body(u, _):
        for b in range(2):
            i = u * 2 + b
            nxt = jnp.minimum(i + 2, last)
            wait_g(b)
            stash_rel(b)
            fire_idx(nxt, b)
            compute(i, b)
            wait_idx(nxt, b)
            fire_g(b)
        return 0

    lax.fori_loop(0, _NCHUNK // 2, pair_body, 0)
    wait_g(0)
    wait_g(1)
    pltpu.sync_copy(out_v, out_hbm.at[wid])


@jax.jit
def _sc_score(node_embeddings, idx_all, rel_weight):
    mesh = plsc.VectorSubcoreMesh(core_axis_name="c", subcore_axis_name="s")
    kfn = functools.partial(
        pl.kernel,
        mesh=mesh,
        out_type=jax.ShapeDtypeStruct((_NW, _OPAD), jnp.float32),
        scratch_types=[
            pltpu.VMEM_SHARED((_N_NODES, _D), jnp.float32),
            pltpu.VMEM((_N_REL, _D), jnp.float32),
            pltpu.VMEM((56,), jnp.int32),
            pltpu.VMEM((3, _C), jnp.int32),
            pltpu.VMEM((3, _C), jnp.int32),
            pltpu.VMEM((_C, _D), jnp.float32),
            pltpu.VMEM((_C, _D), jnp.float32),
            pltpu.VMEM((_C, _D), jnp.float32),
            pltpu.VMEM((_C, _D), jnp.float32),
            pltpu.VMEM((_OPAD,), jnp.float32),
            pltpu.SemaphoreType.DMA,
            pltpu.SemaphoreType.DMA,
            pltpu.SemaphoreType.DMA,
            pltpu.SemaphoreType.DMA,
            pltpu.SemaphoreType.DMA,
            pltpu.SemaphoreType.DMA,
        ],
    )(_sc_body)
    return kfn(node_embeddings, idx_all, rel_weight)


def kernel(node_embeddings, edge_index, relation_type, rel_weight):
    head = edge_index[0].reshape(_NW, _NCHUNK, _C)
    tail = edge_index[1].reshape(_NW, _NCHUNK, _C)
    rel_type = relation_type.astype(jnp.int32).reshape(_NW, _NCHUNK, _C)
    idx_all = jnp.stack([head, tail, rel_type], axis=2)
    out = _sc_score(node_embeddings, idx_all, rel_weight)
    return out[:, :_EW].reshape(_E)


# speedup vs baseline: 2.4184x; 1.0460x over previous
import functools

import jax
import jax.numpy as jnp
from jax import lax
from jax.experimental import pallas as pl
from jax.experimental.pallas import tpu as pltpu
from jax.experimental.pallas import tpu_sc as plsc

_N_NODES = 10000
_D = 128
_N_REL = 16
_E = 320000

_L = 16
_NC = 2
_NS = 16
_NW = _NC * _NS
_EW = _E // _NW
_C = 64
_NCHUNK = 158
_EWPAD = _NCHUNK * _C
_DCH = _D // _L


def _sc_body(node_hbm, idx_hbm, relw_hbm, out_hbm,
             node_sh,
             rtab_v, relbuf_v, ibuf0, ibuf1,
             hrows0, trows0, hrows1, trows1,
             out_v,
             si0, si1, sh0, st0, sh1, st1):
    wid = lax.axis_index("s") * _NC + lax.axis_index("c")
    sid = lax.axis_index("s")
    lane = lax.iota(jnp.int32, _L)
    perms = [jnp.bitwise_xor(lane, jnp.int32(1 << b)) for b in range(4)]
    dcols = [lane + jnp.int32(dd * _L) for dd in range(_DCH)]
    gdn = lax.GatherDimensionNumbers(
        offset_dims=(), collapsed_slice_dims=(0,), start_index_map=(0,))

    def _permute(x, p):
        return lax.gather(x, p[:, None], gdn, (1,),
                          mode=lax.GatherScatterMode.PROMISE_IN_BOUNDS)

    ibufs = (ibuf0, ibuf1)
    isems = (si0, si1)
    rows = ((hrows0, trows0), (hrows1, trows1))
    gsems = ((sh0, st0), (sh1, st1))

    @pl.when(sid == 0)
    def _():
        pltpu.sync_copy(node_hbm, node_sh)

    pltpu.sync_copy(relw_hbm, rtab_v)
    plsc.subcore_barrier()

    def fire_idx(i, b):
        pltpu.async_copy(idx_hbm.at[wid, i], ibufs[b], isems[b])

    def wait_idx(i, b):
        pltpu.make_async_copy(idx_hbm.at[wid, i], ibufs[b], isems[b]).wait()

    def fire_g(b):
        hr, tr = rows[b]
        sh, st = gsems[b]
        ib = ibufs[b]
        pltpu.async_copy(node_sh.at[ib.at[0]], hr, sh)
        pltpu.async_copy(node_sh.at[ib.at[1]], tr, st)

    def wait_g(b):
        hr, tr = rows[b]
        sh, st = gsems[b]
        ib = ibufs[b]
        pltpu.make_async_copy(node_sh.at[ib.at[0]], hr, sh).wait()
        pltpu.make_async_copy(node_sh.at[ib.at[1]], tr, st).wait()

    def stash_rel(b):
        ib = ibufs[b]
        for s in range(_C // _L):
            relbuf_v[pl.ds(s * _L, _L)] = ib[2, pl.ds(s * _L, _L)]

    def compute(i, b):
        hr, tr = rows[b]

        def quad(nk, base_row, q4, scores):
            rvq = relbuf_v[pl.ds(base_row + q4 * 4, _L)]
            for j in range(nk):
                k = q4 * 4 + j
                row = base_row + k
                rel_e = rvq[j]
                ps = [hr[row, pl.ds(dd * _L, _L)]
                      * rtab_v[rel_e, pl.ds(dd * _L, _L)]
                      * tr[row, pl.ds(dd * _L, _L)]
                      for dd in range(_DCH)]
                while len(ps) > 1:
                    ps = [ps[m] + ps[m + 1] for m in range(0, len(ps), 2)]
                acc = ps[0]
                for p in perms:
                    acc = acc + _permute(acc, p)
                scores = jnp.where(lane == k, acc, scores)
            return scores

        def group_body(g, _):
            scores = lax.fori_loop(
                0, 4, functools.partial(quad, 4, g * _L),
                jnp.zeros((_L,), jnp.float32))
            out_v[pl.ds(i * _C + g * _L, _L)] = scores * (1.0 / _D)
            return 0

        lax.fori_loop(0, _C // _L, group_body, 0)

    last = jnp.int32(_NCHUNK - 1)
    fire_idx(0, 0)
    fire_idx(1, 1)
    wait_idx(0, 0)
    fire_g(0)
    wait_idx(1, 1)
    fire_g(1)

    def pair_body(u, _):
        for b in range(2):
            i = u * 2 + b
            nxt = jnp.minimum(i + 2, last)
            wait_g(b)
            stash_rel(b)
            fire_idx(nxt, b)
            compute(i, b)
            wait_idx(nxt, b)
            fire_g(b)
        return 0

    lax.fori_loop(0, _NCHUNK // 2, pair_body, 0)
    wait_g(0)
    wait_g(1)
    pltpu.sync_copy(out_v, out_hbm.at[wid])


@jax.jit
def _sc_score(node_embeddings, idx_all, rel_weight):
    mesh = plsc.VectorSubcoreMesh(core_axis_name="c", subcore_axis_name="s")
    kfn = functools.partial(
        pl.kernel,
        mesh=mesh,
        out_type=jax.ShapeDtypeStruct((_NW, _EWPAD), jnp.float32),
        scratch_types=[
            pltpu.VMEM_SHARED((_N_NODES, _D), jnp.float32),
            pltpu.VMEM((_N_REL, _D), jnp.float32),
            pltpu.VMEM((_C + _L,), jnp.int32),
            pltpu.VMEM((3, _C), jnp.int32),
            pltpu.VMEM((3, _C), jnp.int32),
            pltpu.VMEM((_C, _D), jnp.float32),
            pltpu.VMEM((_C, _D), jnp.float32),
            pltpu.VMEM((_C, _D), jnp.float32),
            pltpu.VMEM((_C, _D), jnp.float32),
            pltpu.VMEM((_EWPAD,), jnp.float32),
            pltpu.SemaphoreType.DMA,
            pltpu.SemaphoreType.DMA,
            pltpu.SemaphoreType.DMA,
            pltpu.SemaphoreType.DMA,
            pltpu.SemaphoreType.DMA,
            pltpu.SemaphoreType.DMA,
        ],
    )(_sc_body)
    return kfn(node_embeddings, idx_all, rel_weight)


def kernel(node_embeddings, edge_index, relation_type, rel_weight):
    pad = ((0, 0), (0, _EWPAD - _EW))
    head = jnp.pad(edge_index[0].reshape(_NW, _EW), pad)
    tail = jnp.pad(edge_index[1].reshape(_NW, _EW), pad)
    rel_type = jnp.pad(
        relation_type.astype(jnp.int32).reshape(_NW, _EW), pad)
    idx_all = jnp.stack(
        [head.reshape(_NW, _NCHUNK, _C),
         tail.reshape(_NW, _NCHUNK, _C),
         rel_type.reshape(_NW, _NCHUNK, _C)], axis=2)
    out = _sc_score(node_embeddings, idx_all, rel_weight)
    return out[:, :_EW].reshape(_E)
